# Initial kernel scaffold; baseline (speedup 1.0000x reference)
import jax
import jax.numpy as jnp
from jax.experimental import pallas as pl


def kernel(queries, keys, values):
    raise NotImplementedError("write your pallas kernel here")


# speedup vs baseline: 3.4777x; 3.4777x over previous
import math

import numpy as np
import jax
import jax.numpy as jnp
from jax.experimental import pallas as pl
from jax.experimental.pallas import tpu as pltpu

_B, _L, _H, _D = 2, 2048, 12, 64
_BH = _B * _H
_U = min(5 * int(np.ceil(np.log(_L))), _L)
_SCALE = np.float32(1.0 / math.sqrt(_D))

_IDX = np.asarray(jax.random.randint(jax.random.key(42), (_L, _U), 0, _L))
_CNT_T_NP = np.zeros((_L, _L), np.int8)
np.add.at(_CNT_T_NP.T, (np.arange(_L)[:, None], _IDX), 1)
_CNT_T = jnp.asarray(_CNT_T_NP)

_KB = 128
_QB = 256
_PREC = jax.lax.Precision.DEFAULT


def _cumsum_rows(x):
    n = x.shape[0]
    s = 1
    while s < n:
        shifted = jnp.concatenate(
            [jnp.zeros((s, x.shape[1]), x.dtype), x[:n - s, :]], axis=0)
        x = x + shifted
        s *= 2
    return x


def _attn_body(q_ref, k_ref, v_ref, ct_ref, o_ref):
    Qh = q_ref[0]
    Kh = k_ref[0]
    Vh = v_ref[0]

    nq = _L // _QB
    nk = _L // _KB
    m_parts = []
    for qc in range(nq):
        Qc = Qh[qc * _QB:(qc + 1) * _QB, :]

        def kb_step(i, carry, Qc=Qc, qc=qc):
            mmax, msum = carry
            Kc = k_ref[0, pl.ds(i * _KB, _KB), :]
            S = jax.lax.dot_general(
                Kc, Qc, (((1,), (1,)), ((), ())),
                preferred_element_type=jnp.float32, precision=_PREC)
            Cc = ct_ref[pl.ds(i * _KB, _KB),
                        qc * _QB:(qc + 1) * _QB].astype(jnp.float32)
            msum = msum + jnp.sum(S * Cc, axis=0, keepdims=True)
            mmax = jnp.maximum(
                mmax,
                jnp.max(jnp.where(Cc > 0.0, S, -jnp.inf), axis=0,
                        keepdims=True))
            return mmax, msum

        mmax0 = jnp.full((1, _QB), -jnp.inf, jnp.float32)
        msum0 = jnp.zeros((1, _QB), jnp.float32)
        mmax, msum = jax.lax.fori_loop(0, nk, kb_step, (mmax0, msum0))
        m_parts.append(mmax - msum * np.float32(1.0 / _L))
    M = jnp.concatenate(m_parts, axis=1)

    lane_iota = jax.lax.broadcasted_iota(jnp.int32, (1, _L), 1)
    oh_rows = []
    for _ in range(_U):
        m = jnp.max(M)
        qidx = jnp.min(jnp.where(M == m, lane_iota, _L))
        oh_rows.append((lane_iota == qidx).astype(jnp.float32))
        M = jnp.where(lane_iota == qidx, -jnp.inf, M)
    OH = jnp.concatenate(oh_rows, axis=0)

    Qr = jax.lax.dot_general(
        OH, Qh, (((1,), (0,)), ((), ())),
        preferred_element_type=jnp.float32,
        precision=jax.lax.Precision.HIGHEST)
    QK = jax.lax.dot_general(
        Qr, Kh, (((1,), (1,)), ((), ())),
        preferred_element_type=jnp.float32,
        precision=jax.lax.Precision.DEFAULT)
    QK = QK * _SCALE
    mx = jnp.max(QK, axis=-1, keepdims=True)
    ex = jnp.exp(QK - mx)
    attn = ex / jnp.sum(ex, axis=-1, keepdims=True)
    Vc = _cumsum_rows(Vh)
    ctx = jax.lax.dot_general(
        attn, Vc, (((1,), (0,)), ((), ())),
        preferred_element_type=jnp.float32,
        precision=jax.lax.Precision.DEFAULT)
    o_ref[0] = ctx


def kernel(queries, keys, values):
    q3 = jnp.transpose(queries, (0, 2, 1, 3)).reshape(_BH, _L, _D)
    k3 = jnp.transpose(keys, (0, 2, 1, 3)).reshape(_BH, _L, _D)
    v3 = jnp.transpose(values, (0, 2, 1, 3)).reshape(_BH, _L, _D)

    out = pl.pallas_call(
        _attn_body,
        grid=(_BH,),
        in_specs=[
            pl.BlockSpec((1, _L, _D), lambda i: (i, 0, 0)),
            pl.BlockSpec((1, _L, _D), lambda i: (i, 0, 0)),
            pl.BlockSpec((1, _L, _D), lambda i: (i, 0, 0)),
            pl.BlockSpec((_L, _L), lambda i: (0, 0)),
        ],
        out_specs=pl.BlockSpec((1, _U, _D), lambda i: (i, 0, 0)),
        out_shape=jax.ShapeDtypeStruct((_BH, _U, _D), jnp.float32),
        compiler_params=pltpu.CompilerParams(
            dimension_semantics=("arbitrary",)),
    )(q3, k3, v3, _CNT_T)

    return out.reshape(_B, _H, _U, _D)


# speedup vs baseline: 4.5091x; 1.2966x over previous
import math

import numpy as np
import jax
import jax.numpy as jnp
from jax.experimental import pallas as pl
from jax.experimental.pallas import tpu as pltpu

_B, _L, _H, _D = 2, 2048, 12, 64
_BH = _B * _H
_U = min(5 * int(np.ceil(np.log(_L))), _L)
_SCALE = np.float32(1.0 / math.sqrt(_D))

_IDX = np.asarray(jax.random.randint(jax.random.key(42), (_L, _U), 0, _L))
_CNT_T_NP = np.zeros((_L, _L), np.int8)
np.add.at(_CNT_T_NP.T, (np.arange(_L)[:, None], _IDX), 1)
_CF = _CNT_T_NP.astype(np.float32)
_NEG = np.where(_CNT_T_NP > 0, np.float32(0.0), np.float32(-3e38))

_KB = 256
_QB = 256
_PREC = jax.lax.Precision.DEFAULT


def _cumsum_rows(x):
    n = x.shape[0]
    s = 1
    while s < n:
        shifted = jnp.concatenate(
            [jnp.zeros((s, x.shape[1]), x.dtype), x[:n - s, :]], axis=0)
        x = x + shifted
        s *= 2
    return x


def _attn_body(q_ref, k_ref, v_ref, cf_ref, ng_ref, o_ref):
    Qh = q_ref[0]
    Kh = k_ref[0]
    Vh = v_ref[0]

    nq = _L // _QB
    nk = _L // _KB
    m_parts = []
    for qc in range(nq):
        Qc = Qh[qc * _QB:(qc + 1) * _QB, :]

        def kb_step(i, carry, Qc=Qc, qc=qc):
            mmax, msum = carry
            Kc = k_ref[0, pl.ds(i * _KB, _KB), :]
            S = jax.lax.dot_general(
                Kc, Qc, (((1,), (1,)), ((), ())),
                preferred_element_type=jnp.float32, precision=_PREC)
            Cc = cf_ref[pl.ds(i * _KB, _KB), qc * _QB:(qc + 1) * _QB]
            Nc = ng_ref[pl.ds(i * _KB, _KB), qc * _QB:(qc + 1) * _QB]
            msum = msum + jnp.sum(S * Cc, axis=0, keepdims=True)
            mmax = jnp.maximum(
                mmax, jnp.max(S + Nc, axis=0, keepdims=True))
            return mmax, msum

        mmax0 = jnp.full((1, _QB), -jnp.inf, jnp.float32)
        msum0 = jnp.zeros((1, _QB), jnp.float32)
        mmax, msum = jax.lax.fori_loop(0, nk, kb_step, (mmax0, msum0))
        m_parts.append(mmax - msum * np.float32(1.0 / _L))
    M = jnp.concatenate(m_parts, axis=1)

    lane_iota = jax.lax.broadcasted_iota(jnp.int32, (1, _L), 1)
    oh_rows = []
    for _ in range(_U):
        m = jnp.max(M)
        qidx = jnp.min(jnp.where(M == m, lane_iota, _L))
        oh_rows.append((lane_iota == qidx).astype(jnp.float32))
        M = jnp.where(lane_iota == qidx, -jnp.inf, M)
    OH = jnp.concatenate(oh_rows, axis=0)

    Qr = jax.lax.dot_general(
        OH, Qh, (((1,), (0,)), ((), ())),
        preferred_element_type=jnp.float32,
        precision=jax.lax.Precision.HIGHEST)
    QK = jax.lax.dot_general(
        Qr, Kh, (((1,), (1,)), ((), ())),
        preferred_element_type=jnp.float32,
        precision=jax.lax.Precision.DEFAULT)
    QK = QK * _SCALE
    mx = jnp.max(QK, axis=-1, keepdims=True)
    ex = jnp.exp(QK - mx)
    attn = ex / jnp.sum(ex, axis=-1, keepdims=True)
    Vc = _cumsum_rows(Vh)
    ctx = jax.lax.dot_general(
        attn, Vc, (((1,), (0,)), ((), ())),
        preferred_element_type=jnp.float32,
        precision=jax.lax.Precision.DEFAULT)
    o_ref[0] = ctx


def kernel(queries, keys, values):
    q3 = jnp.transpose(queries, (0, 2, 1, 3)).reshape(_BH, _L, _D)
    k3 = jnp.transpose(keys, (0, 2, 1, 3)).reshape(_BH, _L, _D)
    v3 = jnp.transpose(values, (0, 2, 1, 3)).reshape(_BH, _L, _D)

    out = pl.pallas_call(
        _attn_body,
        grid=(_BH,),
        in_specs=[
            pl.BlockSpec((1, _L, _D), lambda i: (i, 0, 0)),
            pl.BlockSpec((1, _L, _D), lambda i: (i, 0, 0)),
            pl.BlockSpec((1, _L, _D), lambda i: (i, 0, 0)),
            pl.BlockSpec((_L, _L), lambda i: (0, 0)),
            pl.BlockSpec((_L, _L), lambda i: (0, 0)),
        ],
        out_specs=pl.BlockSpec((1, _U, _D), lambda i: (i, 0, 0)),
        out_shape=jax.ShapeDtypeStruct((_BH, _U, _D), jnp.float32),
        compiler_params=pltpu.CompilerParams(
            dimension_semantics=("arbitrary",)),
    )(q3, k3, v3, _CF, _NEG)

    return out.reshape(_B, _H, _U, _D)


# speedup vs baseline: 4.5239x; 1.0033x over previous
import math

import numpy as np
import jax
import jax.numpy as jnp
from jax import lax
from jax.experimental import pallas as pl
from jax.experimental.pallas import tpu as pltpu
from jax.experimental.pallas import tpu_sc as plsc

_B, _L, _H, _D = 2, 2048, 12, 64
_BH = _B * _H
_U = min(5 * int(np.ceil(np.log(_L))), _L)
_UP = 48
_SCALE = np.float32(1.0 / math.sqrt(_D))

_IDX = np.asarray(jax.random.randint(jax.random.key(42), (_L, _U), 0, _L))
_CNT_T_NP = np.zeros((_L, _L), np.int8)
np.add.at(_CNT_T_NP.T, (np.arange(_L)[:, None], _IDX), 1)
_CF = _CNT_T_NP.astype(np.float32)
_NEG = np.where(_CNT_T_NP > 0, np.float32(0.0), np.float32(-3e38))

_KB = 256
_QB = 256
_PREC = jax.lax.Precision.DEFAULT


def _mtopk_body(q_ref, k_ref, cf_ref, ng_ref, i_ref):
    Qh = q_ref[0]
    nq = _L // _QB
    nk = _L // _KB
    m_parts = []
    for qc in range(nq):
        Qc = Qh[qc * _QB:(qc + 1) * _QB, :]

        def kb_step(i, carry, Qc=Qc, qc=qc):
            mmax, msum = carry
            Kc = k_ref[0, pl.ds(i * _KB, _KB), :]
            S = jax.lax.dot_general(
                Kc, Qc, (((1,), (1,)), ((), ())),
                preferred_element_type=jnp.float32, precision=_PREC)
            Cc = cf_ref[pl.ds(i * _KB, _KB), qc * _QB:(qc + 1) * _QB]
            Nc = ng_ref[pl.ds(i * _KB, _KB), qc * _QB:(qc + 1) * _QB]
            msum = msum + jnp.sum(S * Cc, axis=0, keepdims=True)
            mmax = jnp.maximum(
                mmax, jnp.max(S + Nc, axis=0, keepdims=True))
            return mmax, msum

        mmax0 = jnp.full((1, _QB), -jnp.inf, jnp.float32)
        msum0 = jnp.zeros((1, _QB), jnp.float32)
        mmax, msum = jax.lax.fori_loop(0, nk, kb_step, (mmax0, msum0))
        m_parts.append(mmax - msum * np.float32(1.0 / _L))
    M = jnp.concatenate(m_parts, axis=1)

    lane_iota = jax.lax.broadcasted_iota(jnp.int32, (1, _L), 1)
    j_iota = jax.lax.broadcasted_iota(jnp.int32, (1, 64), 1)
    acc = jnp.zeros((1, 64), jnp.int32)
    for j in range(_U):
        m = jnp.max(M)
        qidx = jnp.min(jnp.where(M == m, lane_iota, _L))
        acc = acc + jnp.where(j_iota == j, qidx, 0)
        M = jnp.where(lane_iota == qidx, -jnp.inf, M)
    i_ref[0] = acc + pl.program_id(0) * _L


def _sc_gather(i_hbm, q_hbm, out_hbm, idxv, rows, sem):
    wid = lax.axis_index("s") * 2 + lax.axis_index("c")

    @pl.when(wid < _BH)
    def _():
        pltpu.sync_copy(i_hbm.at[wid], idxv)
        pltpu.async_copy(q_hbm.at[idxv], rows, sem).wait()
        pltpu.sync_copy(rows.at[pl.ds(0, _U)], out_hbm.at[wid])


def _cumsum_rows(x):
    n = x.shape[0]
    s = 1
    while s < n:
        shifted = jnp.concatenate(
            [jnp.zeros((s, x.shape[1]), x.dtype), x[:n - s, :]], axis=0)
        x = x + shifted
        s *= 2
    return x


def _ctx_body(qr_ref, k_ref, v_ref, o_ref):
    Qr = qr_ref[0][:, :_D]
    Kh = k_ref[0]
    Vh = v_ref[0]
    QK = jax.lax.dot_general(
        Qr, Kh, (((1,), (1,)), ((), ())),
        preferred_element_type=jnp.float32,
        precision=jax.lax.Precision.DEFAULT)
    QK = QK * _SCALE
    mx = jnp.max(QK, axis=-1, keepdims=True)
    ex = jnp.exp(QK - mx)
    attn = ex / jnp.sum(ex, axis=-1, keepdims=True)
    Vc = _cumsum_rows(Vh)
    ctx = jax.lax.dot_general(
        attn, Vc, (((1,), (0,)), ((), ())),
        preferred_element_type=jnp.float32,
        precision=jax.lax.Precision.DEFAULT)
    o_ref[0] = ctx


def kernel(queries, keys, values):
    q3 = jnp.transpose(queries, (0, 2, 1, 3)).reshape(_BH, _L, _D)
    k3 = jnp.transpose(keys, (0, 2, 1, 3)).reshape(_BH, _L, _D)
    v3 = jnp.transpose(values, (0, 2, 1, 3)).reshape(_BH, _L, _D)

    idx = pl.pallas_call(
        _mtopk_body,
        grid=(_BH,),
        in_specs=[
            pl.BlockSpec((1, _L, _D), lambda i: (i, 0, 0)),
            pl.BlockSpec((1, _L, _D), lambda i: (i, 0, 0)),
            pl.BlockSpec((_L, _L), lambda i: (0, 0)),
            pl.BlockSpec((_L, _L), lambda i: (0, 0)),
        ],
        out_specs=pl.BlockSpec((1, 1, 64), lambda i: (i, 0, 0)),
        out_shape=jax.ShapeDtypeStruct((_BH, 1, 64), jnp.int32),
        compiler_params=pltpu.CompilerParams(
            dimension_semantics=("arbitrary",)),
    )(q3, k3, _CF, _NEG)

    qflat = q3.reshape(_BH * _L, _D)
    q_pad = jnp.concatenate([qflat, jnp.zeros_like(qflat)], axis=1)
    mesh = plsc.VectorSubcoreMesh(core_axis_name="c", subcore_axis_name="s")
    qr = pl.kernel(
        _sc_gather,
        mesh=mesh,
        out_type=jax.ShapeDtypeStruct((_BH, _U, 128), jnp.float32),
        scratch_types=[
            pltpu.VMEM((64,), jnp.int32),
            pltpu.VMEM((64, 128), jnp.float32),
            pltpu.SemaphoreType.DMA,
        ],
    )(idx.reshape(_BH, 64), q_pad)

    out = pl.pallas_call(
        _ctx_body,
        grid=(_BH,),
        in_specs=[
            pl.BlockSpec((1, _U, 128), lambda i: (i, 0, 0)),
            pl.BlockSpec((1, _L, _D), lambda i: (i, 0, 0)),
            pl.BlockSpec((1, _L, _D), lambda i: (i, 0, 0)),
        ],
        out_specs=pl.BlockSpec((1, _U, _D), lambda i: (i, 0, 0)),
        out_shape=jax.ShapeDtypeStruct((_BH, _U, _D), jnp.float32),
        compiler_params=pltpu.CompilerParams(
            dimension_semantics=("arbitrary",)),
    )(qr, k3, v3)

    return out.reshape(_B, _H, _U, _D)
